# Initial kernel scaffold; baseline (speedup 1.0000x reference)
import jax
import jax.numpy as jnp
from jax.experimental import pallas as pl


def kernel(x, index_u, index_shortest_path_distance, batch, W1, b1, W2, b2, eps):
    raise NotImplementedError("write your pallas kernel here")


# speedup vs baseline: 5.2253x; 5.2253x over previous
import functools

import jax
import jax.numpy as jnp
from jax import lax
from jax.experimental import pallas as pl
from jax.experimental.pallas import tpu as pltpu
from jax.experimental.pallas import tpu_sc as plsc

E = 320000
N = 10000
C = 128
G = 64

NC = 2
NS = 16
NW = NC * NS

CHUNK = 128
NCHUNKS = E // CHUNK
ZROWS = 200
NZ = N // ZROWS
GCH = 80
NGCH = N // GCH

_mesh = plsc.VectorSubcoreMesh(core_axis_name="c", subcore_axis_name="s")


def _wid():
    cid = lax.axis_index("c")
    sid = lax.axis_index("s")
    return cid, sid, sid * NC + cid


@functools.partial(
    pl.kernel,
    out_type=jax.ShapeDtypeStruct((NC, N, C), jnp.float32),
    mesh=_mesh,
    scratch_types=[
        pltpu.VMEM((CHUNK,), jnp.int32),
        pltpu.VMEM((CHUNK, C), jnp.float32),
        pltpu.VMEM_SHARED((N, C), jnp.float32),
    ],
)
def _agg_kernel(x_hbm, iu_hbm, zeros_hbm, out_hbm, idx_v, rows_v, agg_sh):
    cid, sid, wid = _wid()

    nz_t = NZ // NS + jnp.where(sid < (NZ % NS), 1, 0)

    def zbody(k, _):
        r0 = (sid + k * NS) * ZROWS
        pltpu.sync_copy(zeros_hbm, agg_sh.at[pl.ds(r0, ZROWS)])
        return 0

    lax.fori_loop(0, nz_t, zbody, 0)
    plsc.subcore_barrier()

    nch = 2500 // NW + jnp.where(wid < (NCHUNKS % NW), 1, 0)

    def body(k, _):
        c = wid + k * NW
        e0 = c * CHUNK
        pltpu.sync_copy(iu_hbm.at[pl.ds(e0, CHUNK)], idx_v)
        pltpu.sync_copy(x_hbm.at[pl.ds(e0, CHUNK)], rows_v)
        pltpu.sync_copy(rows_v, agg_sh.at[idx_v], add=True)
        return 0

    lax.fori_loop(0, nch, body, 0)
    plsc.subcore_barrier()

    def wbody(k, _):
        r0 = (sid + k * NS) * ZROWS
        pltpu.sync_copy(agg_sh.at[pl.ds(r0, ZROWS)], out_hbm.at[cid, pl.ds(r0, ZROWS)])
        return 0

    lax.fori_loop(0, nz_t, wbody, 0)


@functools.partial(
    pl.kernel,
    out_type=jax.ShapeDtypeStruct((N, C), jnp.float32),
    mesh=_mesh,
    scratch_types=[
        pltpu.VMEM((GCH,), jnp.int32),
        pltpu.VMEM((GCH, C), jnp.float32),
        pltpu.SemaphoreType.DMA,
    ],
)
def _gather_kernel(x_hbm, ispd_hbm, out_hbm, idx_v, rows_v, sem):
    cid, sid, wid = _wid()
    nch = NGCH // NW + jnp.where(wid < (NGCH % NW), 1, 0)

    def body(k, _):
        r0 = (wid + k * NW) * GCH
        pltpu.sync_copy(ispd_hbm.at[pl.ds(r0, GCH)], idx_v)
        pltpu.async_copy(x_hbm.at[idx_v], rows_v, sem).wait()
        pltpu.sync_copy(rows_v, out_hbm.at[pl.ds(r0, GCH)])
        return 0

    lax.fori_loop(0, nch, body, 0)


def _mlp_body(h0_ref, a0_ref, a1_ref, eps_ref, w1_ref, b1_ref, w2_ref, b2_ref, out_ref):
    scale = 1.0 + eps_ref[0, 0]
    hin = h0_ref[...] * scale + a0_ref[...] + a1_ref[...]
    h1 = jnp.dot(hin, w1_ref[...], preferred_element_type=jnp.float32) + b1_ref[...]
    h1 = jnp.maximum(h1, 0.0)
    h2 = jnp.dot(h1, w2_ref[...], preferred_element_type=jnp.float32) + b2_ref[...]
    out_ref[...] = jnp.maximum(h2, 0.0)


_MLP_R = 1000


def _mlp(h0, a0, a1, eps, w1t, b1, w2t, b2):
    grid = (N // _MLP_R,)
    return pl.pallas_call(
        _mlp_body,
        grid=grid,
        in_specs=[
            pl.BlockSpec((_MLP_R, C), lambda i: (i, 0)),
            pl.BlockSpec((_MLP_R, C), lambda i: (i, 0)),
            pl.BlockSpec((_MLP_R, C), lambda i: (i, 0)),
            pl.BlockSpec((1, 1), lambda i: (0, 0)),
            pl.BlockSpec((C, C), lambda i: (0, 0)),
            pl.BlockSpec((1, C), lambda i: (0, 0)),
            pl.BlockSpec((C, C), lambda i: (0, 0)),
            pl.BlockSpec((1, C), lambda i: (0, 0)),
        ],
        out_specs=pl.BlockSpec((_MLP_R, C), lambda i: (i, 0)),
        out_shape=jax.ShapeDtypeStruct((N, C), jnp.float32),
    )(h0, a0, a1, eps, w1t, b1, w2t, b2)


@functools.partial(
    pl.kernel,
    out_type=jax.ShapeDtypeStruct((NC, G, C), jnp.float32),
    mesh=_mesh,
    scratch_types=[
        pltpu.VMEM((CHUNK,), jnp.int32),
        pltpu.VMEM((CHUNK,), jnp.int32),
        pltpu.VMEM((CHUNK, C), jnp.float32),
        pltpu.VMEM_SHARED((G, C), jnp.float32),
        pltpu.SemaphoreType.DMA,
    ],
)
def _pool_kernel(h_hbm, iu_hbm, batch_hbm, zeros_hbm, out_hbm, idxu_v, idxb_v, rows_v, acc_sh, sem):
    cid, sid, wid = _wid()

    @pl.when(sid == 0)
    def _():
        pltpu.sync_copy(zeros_hbm.at[pl.ds(0, G)], acc_sh)

    plsc.subcore_barrier()
    nch = NCHUNKS // NW + jnp.where(wid < (NCHUNKS % NW), 1, 0)

    def body(k, _):
        e0 = (wid + k * NW) * CHUNK
        pltpu.sync_copy(iu_hbm.at[pl.ds(e0, CHUNK)], idxu_v)
        pltpu.sync_copy(batch_hbm.at[pl.ds(e0, CHUNK)], idxb_v)
        pltpu.async_copy(h_hbm.at[idxu_v], rows_v, sem).wait()
        pltpu.sync_copy(rows_v, acc_sh.at[idxb_v], add=True)
        return 0

    lax.fori_loop(0, nch, body, 0)
    plsc.subcore_barrier()

    @pl.when(sid == 0)
    def _():
        pltpu.sync_copy(acc_sh, out_hbm.at[cid])


def _combine_body(p_ref, out_ref):
    out_ref[...] = p_ref[0] + p_ref[1]


def _combine(p):
    return pl.pallas_call(
        _combine_body,
        out_shape=jax.ShapeDtypeStruct((G, C), jnp.float32),
    )(p)


def kernel(x, index_u, index_shortest_path_distance, batch, W1, b1, W2, b2, eps):
    zeros = jnp.zeros((ZROWS, C), jnp.float32)
    agg2 = _agg_kernel(x, index_u, zeros)
    h0 = _gather_kernel(x, index_shortest_path_distance)
    h = _mlp(h0, agg2[0], agg2[1], eps.reshape(1, 1), W1.T, b1.reshape(1, C),
             W2.T, b2.reshape(1, C))
    pool2 = _pool_kernel(h, index_u, batch, zeros)
    return _combine(pool2)


# speedup vs baseline: 6.8507x; 1.3111x over previous
import functools

import jax
import jax.numpy as jnp
from jax import lax
from jax.experimental import pallas as pl
from jax.experimental.pallas import tpu as pltpu
from jax.experimental.pallas import tpu_sc as plsc

E = 320000
N = 10000
C = 128
G = 64

NC = 2
NS = 16
NW = NC * NS

CHUNK = 128
NCHUNKS = E // CHUNK
ZROWS = 200
NZ = N // ZROWS
GCH = 80
NGCH = N // GCH

_mesh = plsc.VectorSubcoreMesh(core_axis_name="c", subcore_axis_name="s")


def _wid():
    cid = lax.axis_index("c")
    sid = lax.axis_index("s")
    return cid, sid, sid * NC + cid


@functools.partial(
    pl.kernel,
    out_type=jax.ShapeDtypeStruct((NC, N, C), jnp.float32),
    mesh=_mesh,
    scratch_types=[
        pltpu.VMEM((CHUNK,), jnp.int32),
        pltpu.VMEM((CHUNK, C), jnp.float32),
        pltpu.VMEM_SHARED((N, C), jnp.float32),
    ],
)
def _agg_kernel(x_hbm, iu_hbm, zeros_hbm, out_hbm, idx_v, rows_v, agg_sh):
    cid, sid, wid = _wid()

    nz_t = NZ // NS + jnp.where(sid < (NZ % NS), 1, 0)

    def zbody(k, _):
        r0 = (sid + k * NS) * ZROWS
        pltpu.sync_copy(zeros_hbm, agg_sh.at[pl.ds(r0, ZROWS)])
        return 0

    lax.fori_loop(0, nz_t, zbody, 0)
    plsc.subcore_barrier()

    nch = 2500 // NW + jnp.where(wid < (NCHUNKS % NW), 1, 0)

    def body(k, _):
        c = wid + k * NW
        e0 = c * CHUNK
        pltpu.sync_copy(iu_hbm.at[pl.ds(e0, CHUNK)], idx_v)
        pltpu.sync_copy(x_hbm.at[pl.ds(e0, CHUNK)], rows_v)
        pltpu.sync_copy(rows_v, agg_sh.at[idx_v], add=True)
        return 0

    lax.fori_loop(0, nch, body, 0)
    plsc.subcore_barrier()

    def wbody(k, _):
        r0 = (sid + k * NS) * ZROWS
        pltpu.sync_copy(agg_sh.at[pl.ds(r0, ZROWS)], out_hbm.at[cid, pl.ds(r0, ZROWS)])
        return 0

    lax.fori_loop(0, nz_t, wbody, 0)


@functools.partial(
    pl.kernel,
    out_type=jax.ShapeDtypeStruct((N, C), jnp.float32),
    mesh=_mesh,
    scratch_types=[
        pltpu.VMEM((GCH,), jnp.int32),
        pltpu.VMEM((GCH, C), jnp.float32),
        pltpu.SemaphoreType.DMA,
    ],
)
def _gather_kernel(x_hbm, ispd_hbm, out_hbm, idx_v, rows_v, sem):
    cid, sid, wid = _wid()
    nch = NGCH // NW + jnp.where(wid < (NGCH % NW), 1, 0)

    def body(k, _):
        r0 = (wid + k * NW) * GCH
        pltpu.sync_copy(ispd_hbm.at[pl.ds(r0, GCH)], idx_v)
        pltpu.async_copy(x_hbm.at[idx_v], rows_v, sem).wait()
        pltpu.sync_copy(rows_v, out_hbm.at[pl.ds(r0, GCH)])
        return 0

    lax.fori_loop(0, nch, body, 0)


def _mlp_body(h0_ref, a0_ref, a1_ref, eps_ref, w1_ref, b1_ref, w2_ref, b2_ref, out_ref):
    scale = 1.0 + eps_ref[0, 0]
    hin = h0_ref[...] * scale + a0_ref[...] + a1_ref[...]
    h1 = jnp.dot(hin, w1_ref[...], preferred_element_type=jnp.float32) + b1_ref[...]
    h1 = jnp.maximum(h1, 0.0)
    h2 = jnp.dot(h1, w2_ref[...], preferred_element_type=jnp.float32) + b2_ref[...]
    out_ref[...] = jnp.maximum(h2, 0.0)


_MLP_R = 1000


def _mlp(h0, a0, a1, eps, w1t, b1, w2t, b2):
    grid = (N // _MLP_R,)
    return pl.pallas_call(
        _mlp_body,
        grid=grid,
        in_specs=[
            pl.BlockSpec((_MLP_R, C), lambda i: (i, 0)),
            pl.BlockSpec((_MLP_R, C), lambda i: (i, 0)),
            pl.BlockSpec((_MLP_R, C), lambda i: (i, 0)),
            pl.BlockSpec((1, 1), lambda i: (0, 0)),
            pl.BlockSpec((C, C), lambda i: (0, 0)),
            pl.BlockSpec((1, C), lambda i: (0, 0)),
            pl.BlockSpec((C, C), lambda i: (0, 0)),
            pl.BlockSpec((1, C), lambda i: (0, 0)),
        ],
        out_specs=pl.BlockSpec((_MLP_R, C), lambda i: (i, 0)),
        out_shape=jax.ShapeDtypeStruct((N, C), jnp.float32),
    )(h0, a0, a1, eps, w1t, b1, w2t, b2)


NB = G * N
ZEL = 2560
NZEL = NB // ZEL


@functools.partial(
    pl.kernel,
    out_type=[
        jax.ShapeDtypeStruct((NB,), jnp.float32),
        jax.ShapeDtypeStruct((NB,), jnp.float32),
    ],
    mesh=_mesh,
    scratch_types=[
        pltpu.VMEM((CHUNK,), jnp.int32),
        pltpu.VMEM((CHUNK,), jnp.int32),
        pltpu.VMEM((CHUNK,), jnp.int32),
        pltpu.VMEM((CHUNK,), jnp.float32),
        pltpu.VMEM_SHARED((NB,), jnp.float32),
    ],
)
def _hist_kernel(iu_hbm, batch_hbm, zeros_hbm, out0_hbm, out1_hbm,
                 idxu_v, idxb_v, flat_v, ones_v, count_sh):
    cid, sid, wid = _wid()

    nzt = NZEL // NS + jnp.where(sid < (NZEL % NS), 1, 0)

    def zbody(k, _):
        pltpu.sync_copy(zeros_hbm, count_sh.at[pl.ds((sid + k * NS) * ZEL, ZEL)])
        return 0

    lax.fori_loop(0, nzt, zbody, 0)
    for j in range(CHUNK // 16):
        ones_v[pl.ds(j * 16, 16)] = jnp.ones((16,), jnp.float32)
    plsc.subcore_barrier()

    nch = NCHUNKS // NW + jnp.where(wid < (NCHUNKS % NW), 1, 0)

    def body(k, _):
        e0 = (wid + k * NW) * CHUNK
        pltpu.sync_copy(iu_hbm.at[pl.ds(e0, CHUNK)], idxu_v)
        pltpu.sync_copy(batch_hbm.at[pl.ds(e0, CHUNK)], idxb_v)
        for j in range(CHUNK // 16):
            sl = pl.ds(j * 16, 16)
            flat_v[sl] = idxu_v[sl] * G + idxb_v[sl]
        pltpu.sync_copy(ones_v, count_sh.at[flat_v], add=True)
        return 0

    lax.fori_loop(0, nch, body, 0)
    plsc.subcore_barrier()

    def wb0(k, _):
        o = (sid + k * NS) * ZEL
        pltpu.sync_copy(count_sh.at[pl.ds(o, ZEL)], out0_hbm.at[pl.ds(o, ZEL)])
        return 0

    def wb1(k, _):
        o = (sid + k * NS) * ZEL
        pltpu.sync_copy(count_sh.at[pl.ds(o, ZEL)], out1_hbm.at[pl.ds(o, ZEL)])
        return 0

    @pl.when(cid == 0)
    def _():
        lax.fori_loop(0, nzt, wb0, 0)

    @pl.when(cid == 1)
    def _():
        lax.fori_loop(0, nzt, wb1, 0)


def _fused_body(h0_ref, a0_ref, a1_ref, c0_ref, c1_ref, eps_ref,
                w1_ref, b1_ref, w2_ref, b2_ref, out_ref):
    i = pl.program_id(0)
    scale = 1.0 + eps_ref[0, 0]
    hin = h0_ref[...] * scale + a0_ref[...] + a1_ref[...]
    h1 = jnp.dot(hin, w1_ref[...], preferred_element_type=jnp.float32) + b1_ref[...]
    h1 = jnp.maximum(h1, 0.0)
    h2 = jnp.dot(h1, w2_ref[...], preferred_element_type=jnp.float32) + b2_ref[...]
    h2 = jnp.maximum(h2, 0.0)
    cnt = c0_ref[...] + c1_ref[...]
    contrib = lax.dot_general(cnt, h2, (((0,), (0,)), ((), ())),
                              preferred_element_type=jnp.float32)

    @pl.when(i == 0)
    def _():
        out_ref[...] = contrib

    @pl.when(i > 0)
    def _():
        out_ref[...] += contrib


def _fused_tc(h0, a0, a1, c0, c1, eps, w1t, b1, w2t, b2):
    grid = (N // _MLP_R,)
    return pl.pallas_call(
        _fused_body,
        grid=grid,
        in_specs=[
            pl.BlockSpec((_MLP_R, C), lambda i: (i, 0)),
            pl.BlockSpec((_MLP_R, C), lambda i: (i, 0)),
            pl.BlockSpec((_MLP_R, C), lambda i: (i, 0)),
            pl.BlockSpec((_MLP_R, G), lambda i: (i, 0)),
            pl.BlockSpec((_MLP_R, G), lambda i: (i, 0)),
            pl.BlockSpec((1, 1), lambda i: (0, 0)),
            pl.BlockSpec((C, C), lambda i: (0, 0)),
            pl.BlockSpec((1, C), lambda i: (0, 0)),
            pl.BlockSpec((C, C), lambda i: (0, 0)),
            pl.BlockSpec((1, C), lambda i: (0, 0)),
        ],
        out_specs=pl.BlockSpec((G, C), lambda i: (0, 0)),
        out_shape=jax.ShapeDtypeStruct((G, C), jnp.float32),
    )(h0, a0, a1, c0, c1, eps, w1t, b1, w2t, b2)


def kernel(x, index_u, index_shortest_path_distance, batch, W1, b1, W2, b2, eps):
    zeros = jnp.zeros((ZROWS, C), jnp.float32)
    zeros1d = jnp.zeros((ZEL,), jnp.float32)
    agg2 = _agg_kernel(x, index_u, zeros)
    h0 = _gather_kernel(x, index_shortest_path_distance)
    c0, c1 = _hist_kernel(index_u, batch, zeros1d)
    return _fused_tc(h0, agg2[0], agg2[1],
                     c0.reshape(N, G), c1.reshape(N, G), eps.reshape(1, 1),
                     W1.T, b1.reshape(1, C), W2.T, b2.reshape(1, C))


# speedup vs baseline: 12.4276x; 1.8141x over previous
import functools

import jax
import jax.numpy as jnp
from jax import lax
from jax.experimental import pallas as pl
from jax.experimental.pallas import tpu as pltpu
from jax.experimental.pallas import tpu_sc as plsc

E = 320000
N = 10000
C = 128
G = 64

NC = 2
NS = 16
NW = NC * NS

CHUNK = 128
NCHUNKS = E // CHUNK
NMAIN = 78
ZROWS = 200
NZ = N // ZROWS
GCH = 80
NGCH = N // GCH
NB = G * N
ZEL = 2560
NZEL = NB // ZEL
HCH = 512
NHCH = E // HCH

_mesh = plsc.VectorSubcoreMesh(core_axis_name="c", subcore_axis_name="s")


@functools.partial(
    pl.kernel,
    out_type=jax.ShapeDtypeStruct((NC, N, C), jnp.float32),
    mesh=_mesh,
    scratch_types=[
        [pltpu.VMEM((CHUNK,), jnp.int32) for _ in range(3)],
        [pltpu.VMEM((CHUNK, C), jnp.float32) for _ in range(3)],
        pltpu.VMEM_SHARED((N, C), jnp.float32),
        [pltpu.SemaphoreType.DMA for _ in range(3)],
        [pltpu.SemaphoreType.DMA for _ in range(3)],
    ],
)
def _agg_kernel(x_hbm, iu_hbm, zrows_hbm, agg_hbm, iu_s, rows_s, agg_sh,
                lsem, ssem):
    cid = lax.axis_index("c")
    sid = lax.axis_index("s")
    wid = sid * NC + cid

    nz_t = NZ // NS + jnp.where(sid < (NZ % NS), 1, 0)

    def zbody(k, _):
        pltpu.sync_copy(zrows_hbm, agg_sh.at[pl.ds((sid + k * NS) * ZROWS, ZROWS)])
        return 0

    lax.fori_loop(0, nz_t, zbody, 0)
    plsc.subcore_barrier()

    def _issue_loads(slot, c):
        e0 = c * CHUNK
        pltpu.async_copy(iu_hbm.at[pl.ds(e0, CHUNK)], iu_s[slot], lsem[slot])
        pltpu.async_copy(x_hbm.at[pl.ds(e0, CHUNK)], rows_s[slot], lsem[slot])

    def _process(slot):
        pltpu.make_async_copy(iu_hbm.at[pl.ds(0, CHUNK)], iu_s[slot], lsem[slot]).wait()
        pltpu.make_async_copy(x_hbm.at[pl.ds(0, CHUNK)], rows_s[slot], lsem[slot]).wait()
        pltpu.async_copy(rows_s[slot], agg_sh.at[iu_s[slot]], ssem[slot], add=True)

    def _wait_scatter(slot):
        pltpu.make_async_copy(rows_s[slot], agg_sh.at[iu_s[slot]], ssem[slot]).wait()

    _issue_loads(0, wid)
    _issue_loads(1, wid + NW)

    def group(g, _):
        for s in range(3):
            k = g * 3 + s
            _process(s)
            s2 = (s + 2) % 3

            @pl.when(k >= 1)
            def _():
                _wait_scatter(s2)

            c2 = wid + (k + 2) * NW

            @pl.when(c2 < NCHUNKS)
            def _():
                _issue_loads(s2, c2)
        return 0

    lax.fori_loop(0, NMAIN // 3, group, 0)
    _wait_scatter(2)

    @pl.when(wid < (NCHUNKS - NMAIN * NW))
    def _():
        _process(0)
        _wait_scatter(0)

    plsc.subcore_barrier()

    def awb(k, _):
        r0 = (sid + k * NS) * ZROWS
        pltpu.sync_copy(agg_sh.at[pl.ds(r0, ZROWS)], agg_hbm.at[cid, pl.ds(r0, ZROWS)])
        return 0

    lax.fori_loop(0, nz_t, awb, 0)


@functools.partial(
    pl.kernel,
    out_type=[
        jax.ShapeDtypeStruct((NB,), jnp.float32),
        jax.ShapeDtypeStruct((NB,), jnp.float32),
        jax.ShapeDtypeStruct((N, C), jnp.float32),
    ],
    mesh=_mesh,
    scratch_types=[
        [pltpu.VMEM((HCH,), jnp.int32) for _ in range(2)],
        [pltpu.VMEM((HCH,), jnp.int32) for _ in range(2)],
        [pltpu.VMEM((CHUNK,), jnp.int32) for _ in range(HCH // CHUNK)],
        pltpu.VMEM((CHUNK,), jnp.float32),
        pltpu.VMEM((GCH,), jnp.int32),
        pltpu.VMEM((GCH, C), jnp.float32),
        pltpu.VMEM_SHARED((NB,), jnp.float32),
        [pltpu.SemaphoreType.DMA for _ in range(2)],
        pltpu.SemaphoreType.DMA,
        pltpu.SemaphoreType.DMA,
    ],
)
def _hist_kernel(x_hbm, iu_hbm, batch_hbm, ispd_hbm, zel_hbm,
                 cnt0_hbm, cnt1_hbm, h0_hbm,
                 iu_s, bt_s, flat_s, ones_v, gidx_v, grow_v, count_sh,
                 lsem, ssem, gsem):
    cid = lax.axis_index("c")
    sid = lax.axis_index("s")
    wid = sid * NC + cid

    nzel_t = NZEL // NS + jnp.where(sid < (NZEL % NS), 1, 0)

    def zbody(k, _):
        pltpu.sync_copy(zel_hbm, count_sh.at[pl.ds((sid + k * NS) * ZEL, ZEL)])
        return 0

    lax.fori_loop(0, nzel_t, zbody, 0)
    for j in range(CHUNK // 16):
        ones_v[pl.ds(j * 16, 16)] = jnp.ones((16,), jnp.float32)
    plsc.subcore_barrier()

    nh_t = NHCH // NW + jnp.where(wid < (NHCH % NW), 1, 0)

    def _issue_loads(slot, c):
        e0 = c * HCH
        pltpu.async_copy(iu_hbm.at[pl.ds(e0, HCH)], iu_s[slot], lsem[slot])
        pltpu.async_copy(batch_hbm.at[pl.ds(e0, HCH)], bt_s[slot], lsem[slot])

    _issue_loads(0, wid)

    @pl.when(1 * NW + wid < NHCH)
    def _():
        _issue_loads(1, wid + NW)

    def visit(k, _):
        for s in range(2):
            ck = 2 * k + s

            @pl.when(ck < nh_t)
            def _():
                pltpu.make_async_copy(iu_hbm.at[pl.ds(0, HCH)], iu_s[s], lsem[s]).wait()
                pltpu.make_async_copy(batch_hbm.at[pl.ds(0, HCH)], bt_s[s], lsem[s]).wait()
                descs = []
                for q in range(HCH // CHUNK):
                    for j in range(CHUNK // 16):
                        sl_src = pl.ds(q * CHUNK + j * 16, 16)
                        sl_dst = pl.ds(j * 16, 16)
                        flat_s[q][sl_dst] = iu_s[s][sl_src] * G + bt_s[s][sl_src]
                for q in range(HCH // CHUNK):
                    descs.append(pltpu.async_copy(
                        ones_v, count_sh.at[flat_s[q]], ssem, add=True))
                for d in descs:
                    d.wait()
                c2 = wid + (ck + 2) * NW

                @pl.when(c2 < NHCH)
                def _():
                    _issue_loads(s, c2)

        return 0

    lax.fori_loop(0, (NHCH // NW + 2) // 2, visit, 0)

    ng_t = NGCH // NW + jnp.where(wid < (NGCH % NW), 1, 0)

    def gbody(k, _):
        r0 = (wid + k * NW) * GCH
        pltpu.sync_copy(ispd_hbm.at[pl.ds(r0, GCH)], gidx_v)
        pltpu.async_copy(x_hbm.at[gidx_v], grow_v, gsem).wait()
        pltpu.sync_copy(grow_v, h0_hbm.at[pl.ds(r0, GCH)])
        return 0

    lax.fori_loop(0, ng_t, gbody, 0)
    plsc.subcore_barrier()

    def cwb0(k, _):
        o = (sid + k * NS) * ZEL
        pltpu.sync_copy(count_sh.at[pl.ds(o, ZEL)], cnt0_hbm.at[pl.ds(o, ZEL)])
        return 0

    def cwb1(k, _):
        o = (sid + k * NS) * ZEL
        pltpu.sync_copy(count_sh.at[pl.ds(o, ZEL)], cnt1_hbm.at[pl.ds(o, ZEL)])
        return 0

    @pl.when(cid == 0)
    def _():
        lax.fori_loop(0, nzel_t, cwb0, 0)

    @pl.when(cid == 1)
    def _():
        lax.fori_loop(0, nzel_t, cwb1, 0)


_MLP_R = 1000


def _fused_body(h0_ref, a0_ref, a1_ref, c0_ref, c1_ref, eps_ref,
                w1_ref, b1_ref, w2_ref, b2_ref, out_ref):
    i = pl.program_id(0)
    scale = 1.0 + eps_ref[0, 0]
    hin = h0_ref[...] * scale + a0_ref[...] + a1_ref[...]
    h1 = jnp.dot(hin, w1_ref[...], preferred_element_type=jnp.float32) + b1_ref[...]
    h1 = jnp.maximum(h1, 0.0)
    h2 = jnp.dot(h1, w2_ref[...], preferred_element_type=jnp.float32) + b2_ref[...]
    h2 = jnp.maximum(h2, 0.0)
    cnt = c0_ref[...] + c1_ref[...]
    contrib = lax.dot_general(cnt, h2, (((0,), (0,)), ((), ())),
                              preferred_element_type=jnp.float32)

    @pl.when(i == 0)
    def _():
        out_ref[...] = contrib

    @pl.when(i > 0)
    def _():
        out_ref[...] += contrib


def _fused_tc(h0, a0, a1, c0, c1, eps, w1t, b1, w2t, b2):
    grid = (N // _MLP_R,)
    return pl.pallas_call(
        _fused_body,
        grid=grid,
        in_specs=[
            pl.BlockSpec((_MLP_R, C), lambda i: (i, 0)),
            pl.BlockSpec((_MLP_R, C), lambda i: (i, 0)),
            pl.BlockSpec((_MLP_R, C), lambda i: (i, 0)),
            pl.BlockSpec((_MLP_R, G), lambda i: (i, 0)),
            pl.BlockSpec((_MLP_R, G), lambda i: (i, 0)),
            pl.BlockSpec((1, 1), lambda i: (0, 0)),
            pl.BlockSpec((C, C), lambda i: (0, 0)),
            pl.BlockSpec((1, C), lambda i: (0, 0)),
            pl.BlockSpec((C, C), lambda i: (0, 0)),
            pl.BlockSpec((1, C), lambda i: (0, 0)),
        ],
        out_specs=pl.BlockSpec((G, C), lambda i: (0, 0)),
        out_shape=jax.ShapeDtypeStruct((G, C), jnp.float32),
    )(h0, a0, a1, c0, c1, eps, w1t, b1, w2t, b2)


def kernel(x, index_u, index_shortest_path_distance, batch, W1, b1, W2, b2, eps):
    zrows = jnp.zeros((ZROWS, C), jnp.float32)
    zel = jnp.zeros((ZEL,), jnp.float32)
    agg2 = _agg_kernel(x, index_u, zrows)
    c0, c1, h0 = _hist_kernel(x, index_u, batch,
                              index_shortest_path_distance, zel)
    return _fused_tc(h0, agg2[0], agg2[1],
                     c0.reshape(N, G), c1.reshape(N, G), eps.reshape(1, 1),
                     W1.T, b1.reshape(1, C), W2.T, b2.reshape(1, C))


# speedup vs baseline: 14.2573x; 1.1472x over previous
import functools

import jax
import jax.numpy as jnp
from jax import lax
from jax.experimental import pallas as pl
from jax.experimental.pallas import tpu as pltpu
from jax.experimental.pallas import tpu_sc as plsc

E = 320000
N = 10000
C = 128
G = 64

NC = 2
NS = 16
NW = NC * NS

CHUNK = 128
NCHUNKS = E // CHUNK
NMAIN = 78
ZROWS = 200
NZ = N // ZROWS
GCH = 80
NGCH = N // GCH
NB = G * N
ZEL = 12800
NZEL = NB // ZEL
HCH = 512
NHCH = E // HCH

_mesh = plsc.VectorSubcoreMesh(core_axis_name="c", subcore_axis_name="s")


@functools.partial(
    pl.kernel,
    out_type=jax.ShapeDtypeStruct((NC, N, C), jnp.float32),
    mesh=_mesh,
    scratch_types=[
        [pltpu.VMEM((CHUNK,), jnp.int32) for _ in range(3)],
        [pltpu.VMEM((CHUNK, C), jnp.float32) for _ in range(3)],
        pltpu.VMEM_SHARED((N, C), jnp.float32),
        [pltpu.SemaphoreType.DMA for _ in range(3)],
        [pltpu.SemaphoreType.DMA for _ in range(3)],
    ],
)
def _agg_kernel(x_hbm, iu_hbm, zrows_hbm, agg_hbm, iu_s, rows_s, agg_sh,
                lsem, ssem):
    cid = lax.axis_index("c")
    sid = lax.axis_index("s")
    wid = sid * NC + cid

    nz_t = NZ // NS + jnp.where(sid < (NZ % NS), 1, 0)

    def zbody(k, _):
        pltpu.sync_copy(zrows_hbm, agg_sh.at[pl.ds((sid + k * NS) * ZROWS, ZROWS)])
        return 0

    lax.fori_loop(0, nz_t, zbody, 0)
    plsc.subcore_barrier()

    def _issue_loads(slot, c):
        e0 = c * CHUNK
        pltpu.async_copy(iu_hbm.at[pl.ds(e0, CHUNK)], iu_s[slot], lsem[slot])
        pltpu.async_copy(x_hbm.at[pl.ds(e0, CHUNK)], rows_s[slot], lsem[slot])

    def _process(slot):
        pltpu.make_async_copy(iu_hbm.at[pl.ds(0, CHUNK)], iu_s[slot], lsem[slot]).wait()
        pltpu.make_async_copy(x_hbm.at[pl.ds(0, CHUNK)], rows_s[slot], lsem[slot]).wait()
        pltpu.async_copy(rows_s[slot], agg_sh.at[iu_s[slot]], ssem[slot], add=True)

    def _wait_scatter(slot):
        pltpu.make_async_copy(rows_s[slot], agg_sh.at[iu_s[slot]], ssem[slot]).wait()

    _issue_loads(0, wid)
    _issue_loads(1, wid + NW)

    def group(g, _):
        for s in range(3):
            k = g * 3 + s
            _process(s)
            s2 = (s + 2) % 3

            @pl.when(k >= 1)
            def _():
                _wait_scatter(s2)

            c2 = wid + (k + 2) * NW

            @pl.when(c2 < NCHUNKS)
            def _():
                _issue_loads(s2, c2)
        return 0

    lax.fori_loop(0, NMAIN // 3, group, 0)
    _wait_scatter(2)

    @pl.when(wid < (NCHUNKS - NMAIN * NW))
    def _():
        _process(0)
        _wait_scatter(0)

    plsc.subcore_barrier()

    def awb(k, _):
        r0 = (sid + k * NS) * ZROWS
        pltpu.sync_copy(agg_sh.at[pl.ds(r0, ZROWS)], agg_hbm.at[cid, pl.ds(r0, ZROWS)])
        return 0

    lax.fori_loop(0, nz_t, awb, 0)


@functools.partial(
    pl.kernel,
    out_type=[
        jax.ShapeDtypeStruct((NB,), jnp.float32),
        jax.ShapeDtypeStruct((NB,), jnp.float32),
        jax.ShapeDtypeStruct((N, C), jnp.float32),
    ],
    mesh=_mesh,
    scratch_types=[
        [pltpu.VMEM((HCH,), jnp.int32) for _ in range(2)],
        [pltpu.VMEM((HCH,), jnp.int32) for _ in range(2)],
        [[pltpu.VMEM((CHUNK,), jnp.int32) for _ in range(HCH // CHUNK)]
         for _ in range(2)],
        pltpu.VMEM((CHUNK,), jnp.float32),
        [pltpu.VMEM((GCH,), jnp.int32) for _ in range(4)],
        [pltpu.VMEM((GCH, C), jnp.float32) for _ in range(4)],
        pltpu.VMEM_SHARED((NB,), jnp.float32),
        [pltpu.SemaphoreType.DMA for _ in range(2)],
        [pltpu.SemaphoreType.DMA for _ in range(2)],
        pltpu.SemaphoreType.DMA,
        pltpu.SemaphoreType.DMA,
        pltpu.SemaphoreType.DMA,
    ],
)
def _hist_kernel(x_hbm, iu_hbm, batch_hbm, ispd_hbm, zel_hbm,
                 cnt0_hbm, cnt1_hbm, h0_hbm,
                 iu_s, bt_s, flat_s, ones_v, gidx_s, grow_s, count_sh,
                 lsem, ssem, gisem, grsem, gosem):
    cid = lax.axis_index("c")
    sid = lax.axis_index("s")
    wid = sid * NC + cid

    nzel_t = NZEL // NS + jnp.where(sid < (NZEL % NS), 1, 0)

    def zbody(k, _):
        pltpu.sync_copy(zel_hbm, count_sh.at[pl.ds((sid + k * NS) * ZEL, ZEL)])
        return 0

    lax.fori_loop(0, nzel_t, zbody, 0)
    for j in range(CHUNK // 16):
        ones_v[pl.ds(j * 16, 16)] = jnp.ones((16,), jnp.float32)
    plsc.subcore_barrier()

    nh_t = NHCH // NW + jnp.where(wid < (NHCH % NW), 1, 0)

    def _issue_loads(slot, c):
        e0 = c * HCH
        pltpu.async_copy(iu_hbm.at[pl.ds(e0, HCH)], iu_s[slot], lsem[slot])
        pltpu.async_copy(batch_hbm.at[pl.ds(e0, HCH)], bt_s[slot], lsem[slot])

    def _wait_scats(slot):
        for q in range(HCH // CHUNK):
            pltpu.make_async_copy(
                ones_v, count_sh.at[flat_s[slot][q]], ssem[slot]).wait()

    _issue_loads(0, wid)

    @pl.when(NW + wid < NHCH)
    def _():
        _issue_loads(1, wid + NW)

    def visit(k, _):
        for s in range(2):
            ck = 2 * k + s

            @pl.when(ck < nh_t)
            def _():
                @pl.when(ck >= 2)
                def _():
                    _wait_scats(s)

                pltpu.make_async_copy(iu_hbm.at[pl.ds(0, HCH)], iu_s[s], lsem[s]).wait()
                pltpu.make_async_copy(batch_hbm.at[pl.ds(0, HCH)], bt_s[s], lsem[s]).wait()
                for q in range(HCH // CHUNK):
                    for j in range(CHUNK // 16):
                        sl_src = pl.ds(q * CHUNK + j * 16, 16)
                        sl_dst = pl.ds(j * 16, 16)
                        flat_s[s][q][sl_dst] = iu_s[s][sl_src] * G + bt_s[s][sl_src]
                for q in range(HCH // CHUNK):
                    pltpu.async_copy(ones_v, count_sh.at[flat_s[s][q]],
                                     ssem[s], add=True)
                c2 = wid + (ck + 2) * NW

                @pl.when(c2 < NHCH)
                def _():
                    _issue_loads(s, c2)

        return 0

    lax.fori_loop(0, (NHCH // NW + 2) // 2, visit, 0)
    _wait_scats(0)
    _wait_scats(1)

    ng_t = NGCH // NW + jnp.where(wid < (NGCH % NW), 1, 0)
    for k in range(4):
        @pl.when(k < ng_t)
        def _(k=k):
            r0 = (wid + k * NW) * GCH
            pltpu.async_copy(ispd_hbm.at[pl.ds(r0, GCH)], gidx_s[k], gisem)

    for k in range(4):
        @pl.when(k < ng_t)
        def _(k=k):
            pltpu.make_async_copy(ispd_hbm.at[pl.ds(0, GCH)], gidx_s[k], gisem).wait()
            pltpu.async_copy(x_hbm.at[gidx_s[k]], grow_s[k], grsem)

    for k in range(4):
        @pl.when(k < ng_t)
        def _(k=k):
            r0 = (wid + k * NW) * GCH
            pltpu.make_async_copy(x_hbm.at[pl.ds(0, GCH)], grow_s[k], grsem).wait()
            pltpu.async_copy(grow_s[k], h0_hbm.at[pl.ds(r0, GCH)], gosem)

    for k in range(4):
        @pl.when(k < ng_t)
        def _(k=k):
            r0 = (wid + k * NW) * GCH
            pltpu.make_async_copy(grow_s[k], h0_hbm.at[pl.ds(r0, GCH)], gosem).wait()

    plsc.subcore_barrier()

    def cwb0(k, _):
        o = (sid + k * NS) * ZEL
        pltpu.sync_copy(count_sh.at[pl.ds(o, ZEL)], cnt0_hbm.at[pl.ds(o, ZEL)])
        return 0

    def cwb1(k, _):
        o = (sid + k * NS) * ZEL
        pltpu.sync_copy(count_sh.at[pl.ds(o, ZEL)], cnt1_hbm.at[pl.ds(o, ZEL)])
        return 0

    @pl.when(cid == 0)
    def _():
        lax.fori_loop(0, nzel_t, cwb0, 0)

    @pl.when(cid == 1)
    def _():
        lax.fori_loop(0, nzel_t, cwb1, 0)


_MLP_R = 2000


def _fused_body(h0_ref, a0_ref, a1_ref, c0_ref, c1_ref, eps_ref,
                w1_ref, b1_ref, w2_ref, b2_ref, out_ref):
    i = pl.program_id(0)
    scale = 1.0 + eps_ref[0, 0]
    hin = h0_ref[...] * scale + a0_ref[...] + a1_ref[...]
    h1 = jnp.dot(hin, w1_ref[...], preferred_element_type=jnp.float32) + b1_ref[...]
    h1 = jnp.maximum(h1, 0.0)
    h2 = jnp.dot(h1, w2_ref[...], preferred_element_type=jnp.float32) + b2_ref[...]
    h2 = jnp.maximum(h2, 0.0)
    cnt = c0_ref[...] + c1_ref[...]
    contrib = lax.dot_general(cnt, h2, (((0,), (0,)), ((), ())),
                              preferred_element_type=jnp.float32)

    @pl.when(i == 0)
    def _():
        out_ref[...] = contrib

    @pl.when(i > 0)
    def _():
        out_ref[...] += contrib


def _fused_tc(h0, a0, a1, c0, c1, eps, w1t, b1, w2t, b2):
    grid = (N // _MLP_R,)
    return pl.pallas_call(
        _fused_body,
        grid=grid,
        in_specs=[
            pl.BlockSpec((_MLP_R, C), lambda i: (i, 0)),
            pl.BlockSpec((_MLP_R, C), lambda i: (i, 0)),
            pl.BlockSpec((_MLP_R, C), lambda i: (i, 0)),
            pl.BlockSpec((_MLP_R, G), lambda i: (i, 0)),
            pl.BlockSpec((_MLP_R, G), lambda i: (i, 0)),
            pl.BlockSpec((1, 1), lambda i: (0, 0)),
            pl.BlockSpec((C, C), lambda i: (0, 0)),
            pl.BlockSpec((1, C), lambda i: (0, 0)),
            pl.BlockSpec((C, C), lambda i: (0, 0)),
            pl.BlockSpec((1, C), lambda i: (0, 0)),
        ],
        out_specs=pl.BlockSpec((G, C), lambda i: (0, 0)),
        out_shape=jax.ShapeDtypeStruct((G, C), jnp.float32),
    )(h0, a0, a1, c0, c1, eps, w1t, b1, w2t, b2)


def kernel(x, index_u, index_shortest_path_distance, batch, W1, b1, W2, b2, eps):
    zrows = jnp.zeros((ZROWS, C), jnp.float32)
    zel = jnp.zeros((ZEL,), jnp.float32)
    agg2 = _agg_kernel(x, index_u, zrows)
    c0, c1, h0 = _hist_kernel(x, index_u, batch,
                              index_shortest_path_distance, zel)
    return _fused_tc(h0, agg2[0], agg2[1],
                     c0.reshape(N, G), c1.reshape(N, G), eps.reshape(1, 1),
                     W1.T, b1.reshape(1, C), W2.T, b2.reshape(1, C))
